# baseline (speedup 1.0000x reference)
import functools

import jax
import jax.numpy as jnp
from jax import lax
from jax.experimental import pallas as pl
from jax.experimental.pallas import tpu as pltpu
from jax.experimental.pallas import tpu_sc as plsc

_TI = 512
_TJ = 4096


def _pairmin(lov, loi, hiv, hii):
    upd = hiv < lov
    return jnp.where(upd, hiv, lov), jnp.where(upd, hii, loi)


def _argmin_body(pred4_ref, tgt4_ref, fwd_ref, bwd_ref, bval_scr, bidx_scr):
    b = pl.program_id(0)
    NP = pred4_ref.shape[1]
    NT = tgt4_ref.shape[2]
    INF = jnp.float32(jnp.inf)
    BIG = jnp.int32(2**30)
    t_off = b * NT
    p_off = b * NP
    NR = _TI // 8

    bval_scr[...] = jnp.full((8, NT), INF, jnp.float32)
    bidx_scr[...] = jnp.zeros((8, NT), jnp.int32)

    lane_iota = lax.broadcasted_iota(jnp.int32, (_TI, 128), 1)
    sub_iota = lax.broadcasted_iota(jnp.int32, (8, _TJ), 0)

    def i_body(ii, _):
        i0 = ii * _TI
        pq = pred4_ref[0, pl.ds(i0, _TI), :]
        px, py, pz, po = pq[:, 0:1], pq[:, 1:2], pq[:, 2:3], pq[:, 3:4]
        p_valid = ((jnp.abs(px) + jnp.abs(py) + jnp.abs(pz)) > 1e-6) | (
            jnp.abs(po) > 1e-6)
        a2 = px * px + py * py + pz * pz
        pxb = px.astype(jnp.bfloat16).astype(jnp.float32)
        pyb = py.astype(jnp.bfloat16).astype(jnp.float32)
        pzb = pz.astype(jnp.bfloat16).astype(jnp.float32)

        def j_body(jj, carry):
            fval, fidx = carry
            j0 = jj * _TJ
            tq = tgt4_ref[0, :, pl.ds(j0, _TJ)]
            tx, ty, tz, to = tq[0:1, :], tq[1:2, :], tq[2:3, :], tq[3:4, :]
            t_valid = ((jnp.abs(tx) + jnp.abs(ty) + jnp.abs(tz)) > 1e-6) | (
                jnp.abs(to) > 1e-6)
            b2 = tx * tx + ty * ty + tz * tz
            txb = 2.0 * tx.astype(jnp.bfloat16).astype(jnp.float32)
            tyb = 2.0 * ty.astype(jnp.bfloat16).astype(jnp.float32)
            tzb = 2.0 * tz.astype(jnp.bfloat16).astype(jnp.float32)
            ab2 = pxb * txb + pyb * tyb + pzb * tzb
            d2 = (a2 + b2) - ab2

            d2f = jnp.where(t_valid, d2, INF)
            for g in range(_TJ // 128):
                vg = d2f[:, 128 * g:128 * (g + 1)]
                jg = lane_iota + (j0 + 128 * g)
                fval, fidx = _pairmin(fval, fidx, vg, jg)

            d2b = jnp.where(p_valid, d2, INF)
            vals = [d2b[8 * r:8 * (r + 1), :] for r in range(NR)]
            idxs = [sub_iota + (i0 + 8 * r) for r in range(NR)]
            while len(vals) > 1:
                nv, ni = [], []
                for k in range(0, len(vals), 2):
                    v, i = _pairmin(vals[k], idxs[k], vals[k + 1], idxs[k + 1])
                    nv.append(v)
                    ni.append(i)
                vals, idxs = nv, ni
            cur_v = bval_scr[:, pl.ds(j0, _TJ)]
            cur_i = bidx_scr[:, pl.ds(j0, _TJ)]
            nv, ni = _pairmin(cur_v, cur_i, vals[0], idxs[0])
            bval_scr[:, pl.ds(j0, _TJ)] = nv
            bidx_scr[:, pl.ds(j0, _TJ)] = ni
            return fval, fidx

        fval0 = jnp.full((_TI, 128), INF, jnp.float32)
        fidx0 = jnp.zeros((_TI, 128), jnp.int32)
        fval, fidx = lax.fori_loop(0, NT // _TJ, j_body, (fval0, fidx0))
        m = jnp.min(fval, axis=1, keepdims=True)
        cand = jnp.where(fval == m, fidx, BIG)
        idx = jnp.min(cand, axis=1, keepdims=True)
        fwd_ref[0, pl.ds(i0, _TI), :] = idx + t_off
        return 0

    lax.fori_loop(0, NP // _TI, i_body, 0)

    def fin_body(jj, _):
        j0 = jj * _TJ
        v = bval_scr[:, pl.ds(j0, _TJ)]
        ix = bidx_scr[:, pl.ds(j0, _TJ)]
        rows_v = [v[r:r + 1, :] for r in range(8)]
        m = rows_v[0]
        for r in range(1, 8):
            m = jnp.minimum(m, rows_v[r])
        idx = jnp.full((1, _TJ), BIG, jnp.int32)
        for r in range(8):
            idx = jnp.minimum(idx, jnp.where(rows_v[r] == m,
                                             ix[r:r + 1, :], BIG))
        bwd_ref[0, :, pl.ds(j0, _TJ)] = idx + p_off
        return 0

    lax.fori_loop(0, NT // _TJ, fin_body, 0)


def _nn_indices(pred4, tgt4):
    B, NP, _ = pred4.shape
    NT = tgt4.shape[2]
    return pl.pallas_call(
        _argmin_body,
        grid=(B,),
        in_specs=[
            pl.BlockSpec((1, NP, 4), lambda b: (b, 0, 0)),
            pl.BlockSpec((1, 4, NT), lambda b: (b, 0, 0)),
        ],
        out_specs=[
            pl.BlockSpec((1, NP, 1), lambda b: (b, 0, 0)),
            pl.BlockSpec((1, 1, NT), lambda b: (b, 0, 0)),
        ],
        out_shape=[
            jax.ShapeDtypeStruct((B, NP, 1), jnp.int32),
            jax.ShapeDtypeStruct((B, 1, NT), jnp.int32),
        ],
        scratch_shapes=[
            pltpu.VMEM((8, NT), jnp.float32),
            pltpu.VMEM((8, NT), jnp.int32),
        ],
    )(pred4, tgt4)


def _gather_sc(tpad, fidx, ppad, bidx):
    info = plsc.get_sparse_core_info()
    NC, NS = info.num_cores, info.num_subcores
    NW = NC * NS
    F = fidx.shape[0] // NW
    G = bidx.shape[0] // NW
    D = tpad.shape[1]

    @functools.partial(
        pl.kernel,
        mesh=plsc.VectorSubcoreMesh(core_axis_name="c", subcore_axis_name="s"),
        out_type=[
            jax.ShapeDtypeStruct((fidx.shape[0], D), jnp.float32),
            jax.ShapeDtypeStruct((bidx.shape[0], D), jnp.float32),
        ],
        scratch_types=[
            pltpu.VMEM((F,), jnp.int32),
            pltpu.VMEM((F, D), jnp.float32),
            pltpu.VMEM((G,), jnp.int32),
            pltpu.VMEM((G, D), jnp.float32),
            pltpu.SemaphoreType.DMA,
            pltpu.SemaphoreType.DMA,
        ],
        compiler_params=pltpu.CompilerParams(use_tc_tiling_on_sc=False),
    )
    def gather_k(t_hbm, fidx_hbm, p_hbm, bidx_hbm, tm_hbm, pm_hbm,
                 fidx_v, trows_v, bidx_v, prows_v, sem1, sem2):
        wid = lax.axis_index("s") * NC + lax.axis_index("c")
        fb = wid * F
        gb = wid * G
        pltpu.sync_copy(fidx_hbm.at[pl.ds(fb, F)], fidx_v)
        cp1 = pltpu.async_copy(t_hbm.at[fidx_v], trows_v, sem1)
        pltpu.sync_copy(bidx_hbm.at[pl.ds(gb, G)], bidx_v)
        cp2 = pltpu.async_copy(p_hbm.at[bidx_v], prows_v, sem2)
        cp1.wait()
        cp2.wait()
        pltpu.sync_copy(trows_v, tm_hbm.at[pl.ds(fb, F)])
        pltpu.sync_copy(prows_v, pm_hbm.at[pl.ds(gb, G)])

    return gather_k(tpad, fidx, ppad, bidx)


def _loss_body(p_ref, t_ref, tm_ref, pm_ref, out_ref):
    p = p_ref[0]
    t = t_ref[0]
    tm = tm_ref[0]
    pm = pm_ref[0]

    pv = ((jnp.abs(p[:, 0:1]) + jnp.abs(p[:, 1:2]) + jnp.abs(p[:, 2:3]))
          > 1e-6) | (jnp.abs(p[:, 13:14]) > 1e-6)
    tv = ((jnp.abs(t[:, 0:1]) + jnp.abs(t[:, 1:2]) + jnp.abs(t[:, 2:3]))
          > 1e-6) | (jnp.abs(t[:, 13:14]) > 1e-6)
    pvf = pv.astype(jnp.float32)
    tvf = tv.astype(jnp.float32)
    n_p = jnp.sum(pvf)
    n_t = jnp.sum(tvf)

    def mmse(x, y, maskf, count, dims):
        se = (x - y) ** 2 * maskf
        return jnp.sum(se) / (jnp.maximum(count, 1.0) * dims)

    pos = mmse(p[:, 0:3], tm[:, 0:3], pvf, n_p, 3.0)
    scl = mmse(p[:, 3:6], tm[:, 3:6], pvf, n_p, 3.0)

    qp = p[:, 6:10]
    qt = tm[:, 6:10]
    qpn = qp / jnp.maximum(
        jnp.sqrt(jnp.sum(qp * qp, axis=1, keepdims=True)), 1e-8)
    qtn = qt / jnp.maximum(
        jnp.sqrt(jnp.sum(qt * qt, axis=1, keepdims=True)), 1e-8)
    dot = jnp.abs(jnp.sum(qpn * qtn, axis=1, keepdims=True))
    rot = 1.0 - jnp.sum(dot * pvf) / jnp.maximum(n_p, 1.0)

    col = mmse(p[:, 10:13], tm[:, 10:13], pvf, n_p, 3.0)
    opa = mmse(p[:, 13:14], tm[:, 13:14], pvf, n_p, 1.0)

    cov = (2.0 * mmse(t[:, 0:3], pm[:, 0:3], tvf, n_t, 3.0)
           + 0.5 * mmse(t[:, 3:6], pm[:, 3:6], tvf, n_t, 3.0)
           + 0.5 * mmse(t[:, 10:13], pm[:, 10:13], tvf, n_t, 3.0)
           + 2.0 * mmse(t[:, 13:14], pm[:, 13:14], tvf, n_t, 1.0))

    lane = lax.broadcasted_iota(jnp.int32, (1, 8), 1)
    row = jnp.zeros((1, 8), jnp.float32)
    for k, v in enumerate((pos, scl, rot, col, opa, cov, n_p, n_t)):
        row = jnp.where(lane == k, v, row)
    out_ref[0] = row


def _losses(pred, target, tm, pm):
    B, NP, _ = pred.shape
    NT = target.shape[1]
    return pl.pallas_call(
        _loss_body,
        grid=(B,),
        in_specs=[
            pl.BlockSpec((1, NP, 14), lambda b: (b, 0, 0)),
            pl.BlockSpec((1, NT, 14), lambda b: (b, 0, 0)),
            pl.BlockSpec((1, NP, 16), lambda b: (b, 0, 0)),
            pl.BlockSpec((1, NT, 16), lambda b: (b, 0, 0)),
        ],
        out_specs=pl.BlockSpec((1, 1, 8), lambda b: (b, 0, 0)),
        out_shape=jax.ShapeDtypeStruct((B, 1, 8), jnp.float32),
    )(pred, target, tm, pm)


def kernel(pred, target):
    B, NP, C = pred.shape
    NT = target.shape[1]

    sel = jnp.array([0, 1, 2, 13], dtype=jnp.int32)
    pred4 = pred[:, :, sel]
    tgt4 = jnp.transpose(target[:, :, sel], (0, 2, 1))

    fwd, bwd = _nn_indices(pred4, tgt4)
    fidx = fwd.reshape(B * NP)
    bidx = bwd.reshape(B * NT)

    pad = ((0, 0), (0, 0), (0, 2))
    tpad = jnp.pad(target, pad).reshape(B * NT, 16)
    ppad = jnp.pad(pred, pad).reshape(B * NP, 16)
    tm, pm = _gather_sc(tpad, fidx, ppad, bidx)

    rows = _losses(pred, target,
                   tm.reshape(B, NP, 16), pm.reshape(B, NT, 16))[:, 0, :]

    n_p = rows[:, 6]
    n_t = rows[:, 7]
    ok = (n_t > 0.0) & (n_p > 0.0)
    comp = rows[:, :6]
    w = jnp.array([10.0, 5.0, 2.0, 5.0, 3.0, 1.0], dtype=jnp.float32)
    batch_loss = jnp.sum(comp * w[None, :], axis=1)
    total = jnp.sum(jnp.where(ok, batch_loss, 0.0)) / B

    outs = []
    for k in range(6):
        v = jnp.float32(0.0)
        for b in range(B):
            v = jnp.where(ok[b], comp[b, k], v)
        outs.append(v)
    return (total, outs[0], outs[1], outs[2], outs[3], outs[4], outs[5])


# speedup vs baseline: 1.5320x; 1.5320x over previous
import functools

import jax
import jax.numpy as jnp
from jax import lax
from jax.experimental import pallas as pl
from jax.experimental.pallas import tpu as pltpu
from jax.experimental.pallas import tpu_sc as plsc

_TI = 1024
_TJ = 2048


def _pairmin(lov, loi, hiv, hii):
    upd = hiv < lov
    return jnp.where(upd, hiv, lov), jnp.where(upd, hii, loi)


def _argmin_body(pred4_ref, tgt4_ref, fwd_ref, bwd_ref, bval_scr, bidx_scr):
    b = pl.program_id(0)
    NP = pred4_ref.shape[1]
    NT = tgt4_ref.shape[2]
    INF = jnp.float32(jnp.inf)
    BIG = jnp.int32(2**30)
    t_off = b * NT
    p_off = b * NP
    NR = _TI // 8

    bval_scr[...] = jnp.full((8, NT), INF, jnp.float32)
    bidx_scr[...] = jnp.zeros((8, NT), jnp.int32)

    lane_iota = lax.broadcasted_iota(jnp.int32, (_TI, 128), 1)
    sub_iota = lax.broadcasted_iota(jnp.int32, (8, _TJ), 0)

    def i_body(ii, _):
        i0 = ii * _TI
        pq = pred4_ref[0, pl.ds(i0, _TI), :]
        px, py, pz, po = pq[:, 0:1], pq[:, 1:2], pq[:, 2:3], pq[:, 3:4]
        p_valid = ((jnp.abs(px) + jnp.abs(py) + jnp.abs(pz)) > 1e-6) | (
            jnp.abs(po) > 1e-6)
        a2 = px * px + py * py + pz * pz
        pxb = px.astype(jnp.bfloat16).astype(jnp.float32)
        pyb = py.astype(jnp.bfloat16).astype(jnp.float32)
        pzb = pz.astype(jnp.bfloat16).astype(jnp.float32)

        def j_body(jj, carry):
            fval, fidx = carry
            j0 = jj * _TJ
            tq = tgt4_ref[0, :, pl.ds(j0, _TJ)]
            tx, ty, tz, to = tq[0:1, :], tq[1:2, :], tq[2:3, :], tq[3:4, :]
            t_valid = ((jnp.abs(tx) + jnp.abs(ty) + jnp.abs(tz)) > 1e-6) | (
                jnp.abs(to) > 1e-6)
            b2 = tx * tx + ty * ty + tz * tz
            txb = 2.0 * tx.astype(jnp.bfloat16).astype(jnp.float32)
            tyb = 2.0 * ty.astype(jnp.bfloat16).astype(jnp.float32)
            tzb = 2.0 * tz.astype(jnp.bfloat16).astype(jnp.float32)
            ab2 = pxb * txb + pyb * tyb + pzb * tzb
            d2 = (a2 + b2) - ab2

            d2f = jnp.where(t_valid, d2, INF)
            for g in range(_TJ // 128):
                vg = d2f[:, 128 * g:128 * (g + 1)]
                jg = lane_iota + (j0 + 128 * g)
                fval, fidx = _pairmin(fval, fidx, vg, jg)

            d2b = jnp.where(p_valid, d2, INF)
            vals = [d2b[8 * r:8 * (r + 1), :] for r in range(NR)]
            idxs = [sub_iota + (i0 + 8 * r) for r in range(NR)]
            while len(vals) > 1:
                nv, ni = [], []
                for k in range(0, len(vals), 2):
                    v, i = _pairmin(vals[k], idxs[k], vals[k + 1], idxs[k + 1])
                    nv.append(v)
                    ni.append(i)
                vals, idxs = nv, ni
            cur_v = bval_scr[:, pl.ds(j0, _TJ)]
            cur_i = bidx_scr[:, pl.ds(j0, _TJ)]
            nv, ni = _pairmin(cur_v, cur_i, vals[0], idxs[0])
            bval_scr[:, pl.ds(j0, _TJ)] = nv
            bidx_scr[:, pl.ds(j0, _TJ)] = ni
            return fval, fidx

        fval0 = jnp.full((_TI, 128), INF, jnp.float32)
        fidx0 = jnp.zeros((_TI, 128), jnp.int32)
        fval, fidx = lax.fori_loop(0, NT // _TJ, j_body, (fval0, fidx0))
        m = jnp.min(fval, axis=1, keepdims=True)
        cand = jnp.where(fval == m, fidx, BIG)
        idx = jnp.min(cand, axis=1, keepdims=True)
        fwd_ref[0, pl.ds(i0, _TI), :] = idx + t_off
        return 0

    lax.fori_loop(0, NP // _TI, i_body, 0)

    def fin_body(jj, _):
        j0 = jj * _TJ
        v = bval_scr[:, pl.ds(j0, _TJ)]
        ix = bidx_scr[:, pl.ds(j0, _TJ)]
        rows_v = [v[r:r + 1, :] for r in range(8)]
        m = rows_v[0]
        for r in range(1, 8):
            m = jnp.minimum(m, rows_v[r])
        idx = jnp.full((1, _TJ), BIG, jnp.int32)
        for r in range(8):
            idx = jnp.minimum(idx, jnp.where(rows_v[r] == m,
                                             ix[r:r + 1, :], BIG))
        bwd_ref[0, :, pl.ds(j0, _TJ)] = idx + p_off
        return 0

    lax.fori_loop(0, NT // _TJ, fin_body, 0)


def _nn_indices(pred4, tgt4):
    B, NP, _ = pred4.shape
    NT = tgt4.shape[2]
    return pl.pallas_call(
        _argmin_body,
        grid=(B,),
        in_specs=[
            pl.BlockSpec((1, NP, 4), lambda b: (b, 0, 0)),
            pl.BlockSpec((1, 4, NT), lambda b: (b, 0, 0)),
        ],
        out_specs=[
            pl.BlockSpec((1, NP, 1), lambda b: (b, 0, 0)),
            pl.BlockSpec((1, 1, NT), lambda b: (b, 0, 0)),
        ],
        out_shape=[
            jax.ShapeDtypeStruct((B, NP, 1), jnp.int32),
            jax.ShapeDtypeStruct((B, 1, NT), jnp.int32),
        ],
        scratch_shapes=[
            pltpu.VMEM((8, NT), jnp.float32),
            pltpu.VMEM((8, NT), jnp.int32),
        ],
    )(pred4, tgt4)


def _gather_sc(tpad, fidx, ppad, bidx):
    info = plsc.get_sparse_core_info()
    NC, NS = info.num_cores, info.num_subcores
    NW = NC * NS
    F = fidx.shape[0] // NW
    G = bidx.shape[0] // NW
    D = tpad.shape[1]

    @functools.partial(
        pl.kernel,
        mesh=plsc.VectorSubcoreMesh(core_axis_name="c", subcore_axis_name="s"),
        out_type=[
            jax.ShapeDtypeStruct((fidx.shape[0], D), jnp.float32),
            jax.ShapeDtypeStruct((bidx.shape[0], D), jnp.float32),
        ],
        scratch_types=[
            pltpu.VMEM((F,), jnp.int32),
            pltpu.VMEM((F, D), jnp.float32),
            pltpu.VMEM((G,), jnp.int32),
            pltpu.VMEM((G, D), jnp.float32),
            pltpu.SemaphoreType.DMA,
            pltpu.SemaphoreType.DMA,
        ],
        compiler_params=pltpu.CompilerParams(use_tc_tiling_on_sc=False),
    )
    def gather_k(t_hbm, fidx_hbm, p_hbm, bidx_hbm, tm_hbm, pm_hbm,
                 fidx_v, trows_v, bidx_v, prows_v, sem1, sem2):
        wid = lax.axis_index("s") * NC + lax.axis_index("c")
        fb = wid * F
        gb = wid * G
        pltpu.sync_copy(fidx_hbm.at[pl.ds(fb, F)], fidx_v)
        cp1 = pltpu.async_copy(t_hbm.at[fidx_v], trows_v, sem1)
        pltpu.sync_copy(bidx_hbm.at[pl.ds(gb, G)], bidx_v)
        cp2 = pltpu.async_copy(p_hbm.at[bidx_v], prows_v, sem2)
        cp1.wait()
        cp2.wait()
        pltpu.sync_copy(trows_v, tm_hbm.at[pl.ds(fb, F)])
        pltpu.sync_copy(prows_v, pm_hbm.at[pl.ds(gb, G)])

    return gather_k(tpad, fidx, ppad, bidx)


def _loss_body(p_ref, t_ref, tm_ref, pm_ref, out_ref):
    p = p_ref[0]
    t = t_ref[0]
    tm = tm_ref[0]
    pm = pm_ref[0]

    pv = ((jnp.abs(p[:, 0:1]) + jnp.abs(p[:, 1:2]) + jnp.abs(p[:, 2:3]))
          > 1e-6) | (jnp.abs(p[:, 13:14]) > 1e-6)
    tv = ((jnp.abs(t[:, 0:1]) + jnp.abs(t[:, 1:2]) + jnp.abs(t[:, 2:3]))
          > 1e-6) | (jnp.abs(t[:, 13:14]) > 1e-6)
    pvf = pv.astype(jnp.float32)
    tvf = tv.astype(jnp.float32)
    n_p = jnp.sum(pvf)
    n_t = jnp.sum(tvf)

    def mmse(x, y, maskf, count, dims):
        se = (x - y) ** 2 * maskf
        return jnp.sum(se) / (jnp.maximum(count, 1.0) * dims)

    pos = mmse(p[:, 0:3], tm[:, 0:3], pvf, n_p, 3.0)
    scl = mmse(p[:, 3:6], tm[:, 3:6], pvf, n_p, 3.0)

    qp = p[:, 6:10]
    qt = tm[:, 6:10]
    qpn = qp / jnp.maximum(
        jnp.sqrt(jnp.sum(qp * qp, axis=1, keepdims=True)), 1e-8)
    qtn = qt / jnp.maximum(
        jnp.sqrt(jnp.sum(qt * qt, axis=1, keepdims=True)), 1e-8)
    dot = jnp.abs(jnp.sum(qpn * qtn, axis=1, keepdims=True))
    rot = 1.0 - jnp.sum(dot * pvf) / jnp.maximum(n_p, 1.0)

    col = mmse(p[:, 10:13], tm[:, 10:13], pvf, n_p, 3.0)
    opa = mmse(p[:, 13:14], tm[:, 13:14], pvf, n_p, 1.0)

    cov = (2.0 * mmse(t[:, 0:3], pm[:, 0:3], tvf, n_t, 3.0)
           + 0.5 * mmse(t[:, 3:6], pm[:, 3:6], tvf, n_t, 3.0)
           + 0.5 * mmse(t[:, 10:13], pm[:, 10:13], tvf, n_t, 3.0)
           + 2.0 * mmse(t[:, 13:14], pm[:, 13:14], tvf, n_t, 1.0))

    lane = lax.broadcasted_iota(jnp.int32, (1, 8), 1)
    row = jnp.zeros((1, 8), jnp.float32)
    for k, v in enumerate((pos, scl, rot, col, opa, cov, n_p, n_t)):
        row = jnp.where(lane == k, v, row)
    out_ref[0] = row


def _losses(pred, target, tm, pm):
    B, NP, _ = pred.shape
    NT = target.shape[1]
    return pl.pallas_call(
        _loss_body,
        grid=(B,),
        in_specs=[
            pl.BlockSpec((1, NP, 14), lambda b: (b, 0, 0)),
            pl.BlockSpec((1, NT, 14), lambda b: (b, 0, 0)),
            pl.BlockSpec((1, NP, 16), lambda b: (b, 0, 0)),
            pl.BlockSpec((1, NT, 16), lambda b: (b, 0, 0)),
        ],
        out_specs=pl.BlockSpec((1, 1, 8), lambda b: (b, 0, 0)),
        out_shape=jax.ShapeDtypeStruct((B, 1, 8), jnp.float32),
    )(pred, target, tm, pm)


def kernel(pred, target):
    B, NP, C = pred.shape
    NT = target.shape[1]

    sel = jnp.array([0, 1, 2, 13], dtype=jnp.int32)
    pred4 = pred[:, :, sel]
    tgt4 = jnp.transpose(target[:, :, sel], (0, 2, 1))

    fwd, bwd = _nn_indices(pred4, tgt4)
    fidx = fwd.reshape(B * NP)
    bidx = bwd.reshape(B * NT)

    pad = ((0, 0), (0, 0), (0, 2))
    tpad = jnp.pad(target, pad).reshape(B * NT, 16)
    ppad = jnp.pad(pred, pad).reshape(B * NP, 16)
    tm, pm = _gather_sc(tpad, fidx, ppad, bidx)

    rows = _losses(pred, target,
                   tm.reshape(B, NP, 16), pm.reshape(B, NT, 16))[:, 0, :]

    n_p = rows[:, 6]
    n_t = rows[:, 7]
    ok = (n_t > 0.0) & (n_p > 0.0)
    comp = rows[:, :6]
    w = jnp.array([10.0, 5.0, 2.0, 5.0, 3.0, 1.0], dtype=jnp.float32)
    batch_loss = jnp.sum(comp * w[None, :], axis=1)
    total = jnp.sum(jnp.where(ok, batch_loss, 0.0)) / B

    outs = []
    for k in range(6):
        v = jnp.float32(0.0)
        for b in range(B):
            v = jnp.where(ok[b], comp[b, k], v)
        outs.append(v)
    return (total, outs[0], outs[1], outs[2], outs[3], outs[4], outs[5])


# speedup vs baseline: 1.9199x; 1.2532x over previous
import functools

import jax
import jax.numpy as jnp
from jax import lax
from jax.experimental import pallas as pl
from jax.experimental.pallas import tpu as pltpu
from jax.experimental.pallas import tpu_sc as plsc

_TI = 1024
_TJ = 2048


def _pairmin(lov, loi, hiv, hii):
    upd = hiv < lov
    return jnp.where(upd, hiv, lov), jnp.where(upd, hii, loi)


def _argmin_body(pred4_ref, tgt4_ref, fwd_ref, bwd_ref, bval_scr, bidx_scr):
    b = pl.program_id(0)
    NP = pred4_ref.shape[1]
    NT = tgt4_ref.shape[2]
    INF = jnp.float32(jnp.inf)
    BIG = jnp.int32(2**30)
    t_off = b * NT
    p_off = b * NP
    NR = _TI // 8

    bval_scr[...] = jnp.full((8, NT), INF, jnp.float32)
    bidx_scr[...] = jnp.zeros((8, NT), jnp.int32)

    lane_iota = lax.broadcasted_iota(jnp.int32, (_TI, 128), 1)
    sub_iota = lax.broadcasted_iota(jnp.int32, (8, _TJ), 0)

    def i_body(ii, _):
        i0 = ii * _TI
        pq = pred4_ref[0, pl.ds(i0, _TI), :]
        px, py, pz, po = pq[:, 0:1], pq[:, 1:2], pq[:, 2:3], pq[:, 3:4]
        p_valid = ((jnp.abs(px) + jnp.abs(py) + jnp.abs(pz)) > 1e-6) | (
            jnp.abs(po) > 1e-6)
        a2 = px * px + py * py + pz * pz
        a2m = jnp.where(p_valid, a2, INF)
        pb3 = pq[:, 0:3].astype(jnp.bfloat16)

        def j_body(jj, carry):
            fval, fidx = carry
            j0 = jj * _TJ
            tq = tgt4_ref[0, :, pl.ds(j0, _TJ)]
            tx, ty, tz, to = tq[0:1, :], tq[1:2, :], tq[2:3, :], tq[3:4, :]
            t_valid = ((jnp.abs(tx) + jnp.abs(ty) + jnp.abs(tz)) > 1e-6) | (
                jnp.abs(to) > 1e-6)
            b2 = tx * tx + ty * ty + tz * tz
            b2m = jnp.where(t_valid, b2, INF)
            t3 = tq[0:3, :]
            tb3 = (t3 + t3).astype(jnp.bfloat16)
            ab2 = jax.lax.dot_general(
                pb3, tb3, (((1,), (0,)), ((), ())),
                preferred_element_type=jnp.float32)
            d2f = (a2 + b2m) - ab2
            d2b = (a2m + b2) - ab2

            for g in range(_TJ // 128):
                vg = d2f[:, 128 * g:128 * (g + 1)]
                jg = lane_iota + (j0 + 128 * g)
                fval, fidx = _pairmin(fval, fidx, vg, jg)

            vals = [d2b[8 * r:8 * (r + 1), :] for r in range(NR)]
            idxs = [sub_iota + (i0 + 8 * r) for r in range(NR)]
            while len(vals) > 1:
                nv, ni = [], []
                for k in range(0, len(vals), 2):
                    v, i = _pairmin(vals[k], idxs[k], vals[k + 1], idxs[k + 1])
                    nv.append(v)
                    ni.append(i)
                vals, idxs = nv, ni
            cur_v = bval_scr[:, pl.ds(j0, _TJ)]
            cur_i = bidx_scr[:, pl.ds(j0, _TJ)]
            nv, ni = _pairmin(cur_v, cur_i, vals[0], idxs[0])
            bval_scr[:, pl.ds(j0, _TJ)] = nv
            bidx_scr[:, pl.ds(j0, _TJ)] = ni
            return fval, fidx

        fval0 = jnp.full((_TI, 128), INF, jnp.float32)
        fidx0 = jnp.zeros((_TI, 128), jnp.int32)
        fval, fidx = lax.fori_loop(0, NT // _TJ, j_body, (fval0, fidx0))
        m = jnp.min(fval, axis=1, keepdims=True)
        cand = jnp.where(fval == m, fidx, BIG)
        idx = jnp.min(cand, axis=1, keepdims=True)
        fwd_ref[0, pl.ds(i0, _TI), :] = idx + t_off
        return 0

    lax.fori_loop(0, NP // _TI, i_body, 0)

    def fin_body(jj, _):
        j0 = jj * _TJ
        v = bval_scr[:, pl.ds(j0, _TJ)]
        ix = bidx_scr[:, pl.ds(j0, _TJ)]
        rows_v = [v[r:r + 1, :] for r in range(8)]
        m = rows_v[0]
        for r in range(1, 8):
            m = jnp.minimum(m, rows_v[r])
        idx = jnp.full((1, _TJ), BIG, jnp.int32)
        for r in range(8):
            idx = jnp.minimum(idx, jnp.where(rows_v[r] == m,
                                             ix[r:r + 1, :], BIG))
        bwd_ref[0, :, pl.ds(j0, _TJ)] = idx + p_off
        return 0

    lax.fori_loop(0, NT // _TJ, fin_body, 0)


def _nn_indices(pred4, tgt4):
    B, NP, _ = pred4.shape
    NT = tgt4.shape[2]
    return pl.pallas_call(
        _argmin_body,
        grid=(B,),
        in_specs=[
            pl.BlockSpec((1, NP, 4), lambda b: (b, 0, 0)),
            pl.BlockSpec((1, 4, NT), lambda b: (b, 0, 0)),
        ],
        out_specs=[
            pl.BlockSpec((1, NP, 1), lambda b: (b, 0, 0)),
            pl.BlockSpec((1, 1, NT), lambda b: (b, 0, 0)),
        ],
        out_shape=[
            jax.ShapeDtypeStruct((B, NP, 1), jnp.int32),
            jax.ShapeDtypeStruct((B, 1, NT), jnp.int32),
        ],
        scratch_shapes=[
            pltpu.VMEM((8, NT), jnp.float32),
            pltpu.VMEM((8, NT), jnp.int32),
        ],
    )(pred4, tgt4)


def _gather_sc(tpad, fidx, ppad, bidx):
    info = plsc.get_sparse_core_info()
    NC, NS = info.num_cores, info.num_subcores
    NW = NC * NS
    F = fidx.shape[0] // NW
    G = bidx.shape[0] // NW
    D = tpad.shape[1]

    @functools.partial(
        pl.kernel,
        mesh=plsc.VectorSubcoreMesh(core_axis_name="c", subcore_axis_name="s"),
        out_type=[
            jax.ShapeDtypeStruct((fidx.shape[0], D), jnp.float32),
            jax.ShapeDtypeStruct((bidx.shape[0], D), jnp.float32),
        ],
        scratch_types=[
            pltpu.VMEM((F,), jnp.int32),
            pltpu.VMEM((F, D), jnp.float32),
            pltpu.VMEM((G,), jnp.int32),
            pltpu.VMEM((G, D), jnp.float32),
            pltpu.SemaphoreType.DMA,
            pltpu.SemaphoreType.DMA,
        ],
        compiler_params=pltpu.CompilerParams(use_tc_tiling_on_sc=False),
    )
    def gather_k(t_hbm, fidx_hbm, p_hbm, bidx_hbm, tm_hbm, pm_hbm,
                 fidx_v, trows_v, bidx_v, prows_v, sem1, sem2):
        wid = lax.axis_index("s") * NC + lax.axis_index("c")
        fb = wid * F
        gb = wid * G
        pltpu.sync_copy(fidx_hbm.at[pl.ds(fb, F)], fidx_v)
        cp1 = pltpu.async_copy(t_hbm.at[fidx_v], trows_v, sem1)
        pltpu.sync_copy(bidx_hbm.at[pl.ds(gb, G)], bidx_v)
        cp2 = pltpu.async_copy(p_hbm.at[bidx_v], prows_v, sem2)
        cp1.wait()
        cp2.wait()
        pltpu.sync_copy(trows_v, tm_hbm.at[pl.ds(fb, F)])
        pltpu.sync_copy(prows_v, pm_hbm.at[pl.ds(gb, G)])

    return gather_k(tpad, fidx, ppad, bidx)


def _loss_body(p_ref, t_ref, tm_ref, pm_ref, out_ref):
    p = p_ref[0]
    t = t_ref[0]
    tm = tm_ref[0]
    pm = pm_ref[0]

    pv = ((jnp.abs(p[:, 0:1]) + jnp.abs(p[:, 1:2]) + jnp.abs(p[:, 2:3]))
          > 1e-6) | (jnp.abs(p[:, 13:14]) > 1e-6)
    tv = ((jnp.abs(t[:, 0:1]) + jnp.abs(t[:, 1:2]) + jnp.abs(t[:, 2:3]))
          > 1e-6) | (jnp.abs(t[:, 13:14]) > 1e-6)
    pvf = pv.astype(jnp.float32)
    tvf = tv.astype(jnp.float32)
    n_p = jnp.sum(pvf)
    n_t = jnp.sum(tvf)

    def mmse(x, y, maskf, count, dims):
        se = (x - y) ** 2 * maskf
        return jnp.sum(se) / (jnp.maximum(count, 1.0) * dims)

    pos = mmse(p[:, 0:3], tm[:, 0:3], pvf, n_p, 3.0)
    scl = mmse(p[:, 3:6], tm[:, 3:6], pvf, n_p, 3.0)

    qp = p[:, 6:10]
    qt = tm[:, 6:10]
    qpn = qp / jnp.maximum(
        jnp.sqrt(jnp.sum(qp * qp, axis=1, keepdims=True)), 1e-8)
    qtn = qt / jnp.maximum(
        jnp.sqrt(jnp.sum(qt * qt, axis=1, keepdims=True)), 1e-8)
    dot = jnp.abs(jnp.sum(qpn * qtn, axis=1, keepdims=True))
    rot = 1.0 - jnp.sum(dot * pvf) / jnp.maximum(n_p, 1.0)

    col = mmse(p[:, 10:13], tm[:, 10:13], pvf, n_p, 3.0)
    opa = mmse(p[:, 13:14], tm[:, 13:14], pvf, n_p, 1.0)

    cov = (2.0 * mmse(t[:, 0:3], pm[:, 0:3], tvf, n_t, 3.0)
           + 0.5 * mmse(t[:, 3:6], pm[:, 3:6], tvf, n_t, 3.0)
           + 0.5 * mmse(t[:, 10:13], pm[:, 10:13], tvf, n_t, 3.0)
           + 2.0 * mmse(t[:, 13:14], pm[:, 13:14], tvf, n_t, 1.0))

    lane = lax.broadcasted_iota(jnp.int32, (1, 8), 1)
    row = jnp.zeros((1, 8), jnp.float32)
    for k, v in enumerate((pos, scl, rot, col, opa, cov, n_p, n_t)):
        row = jnp.where(lane == k, v, row)
    out_ref[0] = row


def _losses(pred, target, tm, pm):
    B, NP, _ = pred.shape
    NT = target.shape[1]
    return pl.pallas_call(
        _loss_body,
        grid=(B,),
        in_specs=[
            pl.BlockSpec((1, NP, 14), lambda b: (b, 0, 0)),
            pl.BlockSpec((1, NT, 14), lambda b: (b, 0, 0)),
            pl.BlockSpec((1, NP, 16), lambda b: (b, 0, 0)),
            pl.BlockSpec((1, NT, 16), lambda b: (b, 0, 0)),
        ],
        out_specs=pl.BlockSpec((1, 1, 8), lambda b: (b, 0, 0)),
        out_shape=jax.ShapeDtypeStruct((B, 1, 8), jnp.float32),
    )(pred, target, tm, pm)


def kernel(pred, target):
    B, NP, C = pred.shape
    NT = target.shape[1]

    sel = jnp.array([0, 1, 2, 13], dtype=jnp.int32)
    pred4 = pred[:, :, sel]
    tgt4 = jnp.transpose(target[:, :, sel], (0, 2, 1))

    fwd, bwd = _nn_indices(pred4, tgt4)
    fidx = fwd.reshape(B * NP)
    bidx = bwd.reshape(B * NT)

    pad = ((0, 0), (0, 0), (0, 2))
    tpad = jnp.pad(target, pad).reshape(B * NT, 16)
    ppad = jnp.pad(pred, pad).reshape(B * NP, 16)
    tm, pm = _gather_sc(tpad, fidx, ppad, bidx)

    rows = _losses(pred, target,
                   tm.reshape(B, NP, 16), pm.reshape(B, NT, 16))[:, 0, :]

    n_p = rows[:, 6]
    n_t = rows[:, 7]
    ok = (n_t > 0.0) & (n_p > 0.0)
    comp = rows[:, :6]
    w = jnp.array([10.0, 5.0, 2.0, 5.0, 3.0, 1.0], dtype=jnp.float32)
    batch_loss = jnp.sum(comp * w[None, :], axis=1)
    total = jnp.sum(jnp.where(ok, batch_loss, 0.0)) / B

    outs = []
    for k in range(6):
        v = jnp.float32(0.0)
        for b in range(B):
            v = jnp.where(ok[b], comp[b, k], v)
        outs.append(v)
    return (total, outs[0], outs[1], outs[2], outs[3], outs[4], outs[5])
